# Initial kernel scaffold; baseline (speedup 1.0000x reference)
import jax
import jax.numpy as jnp
from jax.experimental import pallas as pl


def kernel(x, edge_index, W1, b1, W2, b2):
    raise NotImplementedError("write your pallas kernel here")


# speedup vs baseline: 13.2645x; 13.2645x over previous
import functools

import jax
import jax.numpy as jnp
from jax import lax
from jax.experimental import pallas as pl
from jax.experimental.pallas import tpu as pltpu
from jax.experimental.pallas import tpu_sc as plsc

N = 10000
E = 320000
D = 128
NC = 2
NS = 16
NW = NC * NS
EPW = E // NW
CH = 80
NCH = EPW // CH
NPAD = 10240
RPT = NPAD // NS
RPD = NPAD // NS

@functools.cache
def _sc_mesh():
    return plsc.VectorSubcoreMesh(
        core_axis_name="c", subcore_axis_name="s",
        num_cores=NC, num_subcores=NS)


def _hop_body(z_hbm, zeros_hbm, src_hbm, dst_hbm, out_hbm,
              src_v, dst_v, rows_v, sem, acc):
    c = lax.axis_index("c")
    s = lax.axis_index("s")
    wid = s * NC + c
    r0 = s * RPT

    @pl.when(c == 0)
    def _():
        pltpu.sync_copy(z_hbm.at[pl.ds(r0, RPT)], acc.at[pl.ds(r0, RPT)])

    @pl.when(c == 1)
    def _():
        pltpu.sync_copy(zeros_hbm.at[pl.ds(r0, RPT)], acc.at[pl.ds(r0, RPT)])

    pltpu.sync_copy(src_hbm.at[wid], src_v)
    pltpu.sync_copy(dst_hbm.at[wid], dst_v)
    plsc.subcore_barrier()

    def body(j, carry):
        pltpu.async_copy(z_hbm.at[src_v.at[j]], rows_v, sem).wait()
        pltpu.sync_copy(rows_v, acc.at[dst_v.at[j]], add=True)
        return carry

    lax.fori_loop(0, NCH, body, 0)
    plsc.subcore_barrier()
    pltpu.sync_copy(acc.at[pl.ds(r0, RPT)], out_hbm.at[c, pl.ds(r0, RPT)])


@functools.cache
def _hop_kernel():
    return pl.kernel(
        _hop_body,
        out_type=jax.ShapeDtypeStruct((NC, NPAD, D), jnp.float32),
        mesh=_sc_mesh(),
        scratch_types=[
            pltpu.VMEM((NCH, CH), jnp.int32),
            pltpu.VMEM((NCH, CH), jnp.int32),
            pltpu.VMEM((CH, D), jnp.float32),
            pltpu.SemaphoreType.DMA,
            pltpu.VMEM_SHARED((NPAD, D), jnp.float32),
        ],
    )


def _hop(z, zeros, src_r, dst_r):
    return _hop_kernel()(z, zeros, src_r, dst_r)


def _prep_kernel(dp_ref, dinv_ref, dinv2_ref):
    deg = dp_ref[0, :, :1] + dp_ref[1, :, :1]
    dinv = lax.rsqrt(deg)
    dinv_ref[...] = dinv
    dinv2_ref[...] = dinv * dinv


def _tc_prep(degp):
    br = 2048
    return pl.pallas_call(
        _prep_kernel,
        grid=(NPAD // br,),
        in_specs=[pl.BlockSpec((NC, br, D), lambda i: (0, i, 0))],
        out_specs=[pl.BlockSpec((br, 1), lambda i: (i, 0)),
                   pl.BlockSpec((br, 1), lambda i: (i, 0))],
        out_shape=[jax.ShapeDtypeStruct((NPAD, 1), jnp.float32),
                   jax.ShapeDtypeStruct((NPAD, 1), jnp.float32)],
    )(degp)


def _scale_kernel(x_ref, s_ref, o_ref):
    o_ref[...] = x_ref[...] * s_ref[...]


def _tc_scale(x, s):
    br = 2048
    return pl.pallas_call(
        _scale_kernel,
        grid=(NPAD // br,),
        in_specs=[pl.BlockSpec((br, D), lambda i: (i, 0)),
                  pl.BlockSpec((br, 1), lambda i: (i, 0))],
        out_specs=pl.BlockSpec((br, D), lambda i: (i, 0)),
        out_shape=jax.ShapeDtypeStruct((NPAD, D), jnp.float32),
    )(x, s)


def _combine_kernel(p_ref, s_ref, o_ref):
    o_ref[...] = (p_ref[0] + p_ref[1]) * s_ref[...]


def _tc_combine(p, s):
    br = 2048
    return pl.pallas_call(
        _combine_kernel,
        grid=(NPAD // br,),
        in_specs=[pl.BlockSpec((NC, br, D), lambda i: (0, i, 0)),
                  pl.BlockSpec((br, 1), lambda i: (i, 0))],
        out_specs=pl.BlockSpec((br, D), lambda i: (i, 0)),
        out_shape=jax.ShapeDtypeStruct((NPAD, D), jnp.float32),
    )(p, s)


def _layer_kernel(p_ref, s_ref, w_ref, b_ref, o_ref, *, relu, post_scale):
    t = (p_ref[0] + p_ref[1]) * s_ref[...]
    y = jnp.dot(t, w_ref[...], preferred_element_type=jnp.float32) + b_ref[...]
    if relu:
        y = jnp.maximum(y, 0.0)
    if post_scale:
        y = y * s_ref[...]
    o_ref[...] = y


def _tc_layer(p, s, w, b, relu, post_scale):
    br = 2048
    return pl.pallas_call(
        functools.partial(_layer_kernel, relu=relu, post_scale=post_scale),
        grid=(NPAD // br,),
        in_specs=[pl.BlockSpec((NC, br, D), lambda i: (0, i, 0)),
                  pl.BlockSpec((br, 1), lambda i: (i, 0)),
                  pl.BlockSpec((D, D), lambda i: (0, 0)),
                  pl.BlockSpec((1, D), lambda i: (0, 0))],
        out_specs=pl.BlockSpec((br, D), lambda i: (i, 0)),
        out_shape=jax.ShapeDtypeStruct((NPAD, D), jnp.float32),
    )(p, s, w, b)


def kernel(x, edge_index, W1, b1, W2, b2):
    src_r = edge_index[0].reshape(NW, NCH, CH)
    dst_r = edge_index[1].reshape(NW, NCH, CH)
    xp = jnp.pad(x, ((0, NPAD - N), (0, 0)))
    zeros = jnp.zeros((NPAD, D), jnp.float32)
    onesm = jnp.ones((NPAD, D), jnp.float32)
    b1r = b1.reshape(1, D)
    b2r = b2.reshape(1, D)

    degp = _hop(onesm, zeros, src_r, dst_r)
    dinv, dinv2 = _tc_prep(degp)

    u0 = _tc_scale(xp, dinv)
    p = _hop(u0, zeros, src_r, dst_r)
    u2 = _tc_combine(p, dinv2)
    p = _hop(u2, zeros, src_r, dst_r)
    v0 = _tc_layer(p, dinv, W1, b1r, relu=True, post_scale=True)
    p = _hop(v0, zeros, src_r, dst_r)
    v2 = _tc_combine(p, dinv2)
    p = _hop(v2, zeros, src_r, dst_r)
    out = _tc_layer(p, dinv, W2, b2r, relu=False, post_scale=False)
    return out[:N]
